# Initial kernel scaffold; baseline (speedup 1.0000x reference)
import jax
import jax.numpy as jnp
from jax.experimental import pallas as pl


def kernel(x_source, edge_index, W, a):
    raise NotImplementedError("write your pallas kernel here")


# speedup vs baseline: 15.3878x; 15.3878x over previous
import jax
import jax.numpy as jnp
from jax import lax
from jax.experimental import pallas as pl
from jax.experimental.pallas import tpu as pltpu
from jax.experimental.pallas import tpu_sc as plsc

N = 10000
E = 320000
D = 128
NC = 2
NS = 16
NW = NC * NS
E_PER_W = E // NW
BE = 80
NCH = E_PER_W // BE
ROWS_PER_W = N // NS
ZR = 125
DN_PER_W = 640
ND = NS * DN_PER_W


def _tc_prep_body(x_ref, w_ref, a2_ref, m_ref, z_ref, g_ref):
    m = jnp.dot(x_ref[...], w_ref[...], preferred_element_type=jnp.float32)
    m_ref[...] = m
    z = jnp.dot(m, a2_ref[...], preferred_element_type=jnp.float32)
    z_ref[...] = z
    c = jnp.max(z[:, 0]) + jnp.max(z[:, 1])
    c = jnp.where(c >= 0.0, c, 0.2 * c)
    g_ref[...] = jnp.full((8, 128), c, dtype=jnp.float32)


def _sc_edge_body(z_hbm, g_hbm, ei_hbm, ej_hbm, m_hbm, out_hbm, den_hbm,
                  zv, gv, iiv, ijv, rows, wv, zbuf, zdbuf, acc, dn, gsem):
    cid = lax.axis_index("c")
    sid = lax.axis_index("s")
    wid = cid * NS + sid

    pltpu.sync_copy(z_hbm, zv)
    pltpu.sync_copy(g_hbm, gv)

    @pl.loop(0, ZR)
    def _zero_rows(r):
        for v in range(D // 16):
            zbuf[r, pl.ds(v * 16, 16)] = jnp.zeros((16,), jnp.float32)

    @pl.loop(0, DN_PER_W // 16)
    def _zero_den(r):
        zdbuf[pl.ds(r * 16, 16)] = jnp.zeros((16,), jnp.float32)

    row0 = sid * ROWS_PER_W
    for t in range(ROWS_PER_W // ZR):
        pltpu.sync_copy(zbuf, acc.at[pl.ds(row0 + t * ZR, ZR)])
    pltpu.sync_copy(zdbuf, dn.at[pl.ds(sid * DN_PER_W, DN_PER_W)])
    plsc.subcore_barrier()

    gamma = gv[...]
    ebase = wid * E_PER_W

    @pl.loop(0, NCH)
    def _chunk(ch):
        base = ebase + ch * BE
        pltpu.sync_copy(ei_hbm.at[pl.ds(base, BE)], iiv.at[0])
        pltpu.sync_copy(ej_hbm.at[pl.ds(base, BE)], ijv.at[0])
        gather = pltpu.async_copy(m_hbm.at[ijv.at[0]], rows.at[0], gsem)
        for g in range(BE // 16):
            iv = iiv[0, pl.ds(g * 16, 16)]
            jv = ijv[0, pl.ds(g * 16, 16)]
            s = plsc.load_gather(zv, [iv * 2])
            d = plsc.load_gather(zv, [jv * 2 + 1])
            e = s + d
            e = jnp.where(e >= 0.0, e, e * 0.2)
            wv[pl.ds(g * 16, 16)] = jnp.exp(e - gamma)
        gather.wait()

        @pl.loop(0, BE)
        def _scale(k):
            wk = plsc.load_gather(wv, [jnp.full((16,), k, jnp.int32)])
            for v in range(D // 16):
                sl = pl.ds(v * 16, 16)
                rows[0, k, sl] = rows[0, k, sl] * wk

        pltpu.sync_copy(rows.at[0], acc.at[iiv.at[0]], add=True)
        pltpu.sync_copy(wv, dn.at[iiv.at[0]], add=True)

    plsc.subcore_barrier()
    for t in range(ROWS_PER_W // ZR):
        r = row0 + t * ZR
        pltpu.sync_copy(acc.at[pl.ds(r, ZR)], zbuf)
        pltpu.sync_copy(zbuf, out_hbm.at[cid, pl.ds(r, ZR)])
    d0 = sid * DN_PER_W
    pltpu.sync_copy(dn.at[pl.ds(d0, DN_PER_W)], zdbuf)
    pltpu.sync_copy(zdbuf, den_hbm.at[cid, pl.ds(d0, DN_PER_W)])


def _make_sc_kernel():
    mesh = plsc.VectorSubcoreMesh(
        core_axis_name="c", subcore_axis_name="s",
        num_cores=NC, num_subcores=NS)
    return pl.kernel(
        _sc_edge_body,
        out_type=(
            jax.ShapeDtypeStruct((NC, N, D), jnp.float32),
            jax.ShapeDtypeStruct((NC, ND), jnp.float32),
        ),
        mesh=mesh,
        scratch_types=[
            pltpu.VMEM((2 * N,), jnp.float32),
            pltpu.VMEM((16,), jnp.float32),
            pltpu.VMEM((1, BE), jnp.int32),
            pltpu.VMEM((1, BE), jnp.int32),
            pltpu.VMEM((1, BE, D), jnp.float32),
            pltpu.VMEM((BE,), jnp.float32),
            pltpu.VMEM((ZR, D), jnp.float32),
            pltpu.VMEM((DN_PER_W,), jnp.float32),
            pltpu.VMEM_SHARED((N, D), jnp.float32),
            pltpu.VMEM_SHARED((ND,), jnp.float32),
            pltpu.SemaphoreType.DMA,
        ],
        compiler_params=pltpu.CompilerParams(
            use_tc_tiling_on_sc=False, needs_layout_passes=False),
    )


def _tc_combine_body(p_ref, d_ref, o_ref):
    num = p_ref[0] + p_ref[1]
    den = d_ref[0, :N] + d_ref[1, :N]
    safe = jnp.where(den > 0.0, den, 1.0)
    o_ref[...] = jnp.maximum(num / safe[:, None], 0.0)


def kernel(x_source, edge_index, W, a):
    a2 = a.reshape(2, D).T

    m, z, g = pl.pallas_call(
        _tc_prep_body,
        out_shape=(
            jax.ShapeDtypeStruct((N, D), jnp.float32),
            jax.ShapeDtypeStruct((N, 2), jnp.float32),
            jax.ShapeDtypeStruct((8, 128), jnp.float32),
        ),
    )(x_source, W, a2)

    zflat = z.reshape(2 * N)
    g16 = g[0, :16]
    ei = edge_index[0]
    ej = edge_index[1]

    partials, dens = _make_sc_kernel()(zflat, g16, ei, ej, m)

    out = pl.pallas_call(
        _tc_combine_body,
        out_shape=jax.ShapeDtypeStruct((N, D), jnp.float32),
    )(partials, dens)
    return out
